# baseline (speedup 1.0000x reference)
import functools

import jax
import jax.numpy as jnp
from jax import lax
from jax.experimental import pallas as pl
from jax.experimental.pallas import tpu as pltpu
from jax.experimental.pallas import tpu_sc as plsc

B, S, D, A, E = 4, 2048, 512, 64, 4
N = B * S

XBLK = 1024
CH = 512
NW = 32
L = 16


def _k1_body(x_ref, wf_ref, bf_ref, wg_ref, bg_ref, adapt_ref, gate_ref,
             acc_ref):
    i = pl.program_id(0)

    @pl.when(i == 0)
    def _():
        acc_ref[...] = jnp.zeros_like(acc_ref)

    acc_ref[...] += jnp.sum(x_ref[...], axis=0, keepdims=True)

    @pl.when(i == pl.num_programs(0) - 1)
    def _():
        mean = acc_ref[...] * (1.0 / N)
        adapt = lax.dot_general(
            mean, wf_ref[...], (((1,), (1,)), ((), ())),
            preferred_element_type=jnp.float32,
            precision=lax.Precision.HIGHEST) + bf_ref[...]
        logits = lax.dot_general(
            adapt, wg_ref[...], (((1,), (1,)), ((), ())),
            preferred_element_type=jnp.float32,
            precision=lax.Precision.HIGHEST) + bg_ref[...]
        m = jnp.max(logits, axis=-1, keepdims=True)
        eg = jnp.exp(logits - m)
        gate_ref[...] = eg / jnp.sum(eg, axis=-1, keepdims=True)
        adapt_ref[...] = adapt


def _perm(v, idx):
    return lax.gather(
        v, idx[:, None],
        lax.GatherDimensionNumbers(offset_dims=(), collapsed_slice_dims=(0,),
                                   start_index_map=(0,)),
        (1,), mode=lax.GatherScatterMode.PROMISE_IN_BOUNDS)


def _sc_delta_body(tm_ref, cm_ref, ex_ref, adapt_ref,
                   dtm_ref, dcm_ref, dex_ref,
                   buf, tmp, obuf, adapt_v):
    wid = lax.axis_index("s") * 2 + lax.axis_index("c")

    pltpu.sync_copy(adapt_ref, adapt_v)
    a_vs = [adapt_v[pl.ds(16 * i, L)] for i in range(4)]
    iota = lax.iota(jnp.int32, L)
    bfly = [lax.bitwise_xor(iota, k) for k in (8, 4, 2, 1)]
    masks = [iota == rr for rr in range(L)]
    zero = jnp.zeros((L,), jnp.float32)

    def segment(slicer, out_ref, out_base, rows_per_tile):
        base = wid * rows_per_tile

        def chunk_body(k, _):
            r0 = base + k * CH
            pltpu.sync_copy(slicer(r0), buf)

            def group_body(g, _):
                off = g * L
                out = zero
                for rr in range(L):
                    row = off + rr
                    s = (buf[row, pl.ds(0, L)] * a_vs[0]
                         + buf[row, pl.ds(16, L)] * a_vs[1]
                         + buf[row, pl.ds(32, L)] * a_vs[2]
                         + buf[row, pl.ds(48, L)] * a_vs[3])
                    for bidx in bfly:
                        s = s + _perm(s, bidx)
                    out = jnp.where(masks[rr], s, out)
                obuf[pl.ds(g * L, L)] = out
                return 0

            lax.fori_loop(0, CH // L, group_body, 0)
            pltpu.sync_copy(obuf, out_ref.at[pl.ds(out_base + r0, CH)])
            return 0

        lax.fori_loop(0, rows_per_tile // CH, chunk_body, 0)

    segment(lambda r0: tm_ref.at[pl.ds(r0, CH)], dtm_ref, 0, (D * D) // NW)
    segment(lambda r0: cm_ref.at[pl.ds(r0, CH)], dcm_ref, 0, (D * D) // NW)
    for e in range(E):
        segment(lambda r0, e=e: ex_ref.at[e, pl.ds(r0, CH)],
                dex_ref, e * D * D, (D * D) // NW)


def _k2_body(dtm_ref, dcm_ref, dex_ref,
             tmW_ref, tmAb_ref, cmW_ref, cmAb_ref,
             exW_ref, exAb_ref, exb_ref, gate_ref,
             tmb_ref, cmb_ref, wout_ref, bout_ref,
             wmem_ref, bmem_ref, wtime_ref, btime_ref,
             v_ref, c_ref):
    tmw = tmW_ref[...] + tmAb_ref[...] + dtm_ref[...]
    cmw = cmW_ref[...] + cmAb_ref[...] + dcm_ref[...]
    ew = exW_ref[...] + exAb_ref[...] + dex_ref[...]
    acc = gate_ref[0, 0] * ew[0]
    for e in range(1, E):
        acc += gate_ref[0, e] * ew[e]
    gate = gate_ref[...]
    bcomb = lax.dot_general(
        gate, exb_ref[...], (((1,), (0,)), ((), ())),
        preferred_element_type=jnp.float32,
        precision=lax.Precision.HIGHEST)

    def chain(u0, c0):
        u1 = jnp.dot(u0, wout_ref[...],
                     preferred_element_type=jnp.float32,
                     precision=lax.Precision.HIGHEST)
        c1 = c0 + jnp.sum(u0 * bout_ref[...], keepdims=True)[:, :1]
        u2 = jnp.dot(u1, acc,
                     preferred_element_type=jnp.float32,
                     precision=lax.Precision.HIGHEST)
        c2 = c1 + jnp.sum(u1 * bcomb, keepdims=True)[:, :1]
        u3 = jnp.dot(u2, cmw,
                     preferred_element_type=jnp.float32,
                     precision=lax.Precision.HIGHEST)
        c3 = c2 + jnp.sum(u2 * cmb_ref[...], keepdims=True)[:, :1]
        u4 = jnp.dot(u3, tmw,
                     preferred_element_type=jnp.float32,
                     precision=lax.Precision.HIGHEST)
        c4 = c3 + jnp.sum(u3 * tmb_ref[...], keepdims=True)[:, :1]
        return u4, c4

    vm, cm = chain(wmem_ref[...], bmem_ref[...])
    vt, ct = chain(wtime_ref[...], btime_ref[...])
    v_ref[0:1, :] = vm
    v_ref[1:2, :] = vt
    c_ref[:, 0:1] = cm
    c_ref[:, 1:2] = ct


def _k3_body(x_ref, v_ref, c_ref, out_ref):
    out_ref[...] = lax.dot_general(
        x_ref[...], v_ref[...], (((1,), (1,)), ((), ())),
        preferred_element_type=jnp.float32,
        precision=lax.Precision.HIGHEST) + c_ref[...]


@jax.jit
def kernel(x, W_feat, b_feat, tm_W, tm_b, tm_AW, tm_Ab, cm_W, cm_b, cm_AW,
           cm_Ab, ex_W, ex_b, ex_AW, ex_Ab, W_gate, b_gate, W_out, b_out,
           W_mem, b_mem, W_time, b_time):
    xf = x.reshape(N, D)

    adapt, gate = pl.pallas_call(
        _k1_body,
        grid=(N // XBLK,),
        in_specs=[
            pl.BlockSpec((XBLK, D), lambda i: (i, 0)),
            pl.BlockSpec((A, D), lambda i: (0, 0)),
            pl.BlockSpec((1, A), lambda i: (0, 0)),
            pl.BlockSpec((E, A), lambda i: (0, 0)),
            pl.BlockSpec((1, E), lambda i: (0, 0)),
        ],
        out_specs=[
            pl.BlockSpec((1, A), lambda i: (0, 0)),
            pl.BlockSpec((1, E), lambda i: (0, 0)),
        ],
        out_shape=[
            jax.ShapeDtypeStruct((1, A), jnp.float32),
            jax.ShapeDtypeStruct((1, E), jnp.float32),
        ],
        scratch_shapes=[pltpu.VMEM((1, D), jnp.float32)],
    )(xf, W_feat, b_feat.reshape(1, A), W_gate, b_gate.reshape(1, E))

    mesh = plsc.VectorSubcoreMesh(core_axis_name="c", subcore_axis_name="s")
    sc_delta = functools.partial(
        pl.kernel,
        mesh=mesh,
        out_type=[
            jax.ShapeDtypeStruct((D * D,), jnp.float32),
            jax.ShapeDtypeStruct((D * D,), jnp.float32),
            jax.ShapeDtypeStruct((E * D * D,), jnp.float32),
        ],
        scratch_types=[
            pltpu.VMEM((CH, A), jnp.float32),
            pltpu.VMEM((L * L,), jnp.float32),
            pltpu.VMEM((CH,), jnp.float32),
            pltpu.VMEM((A,), jnp.float32),
        ],
    )(_sc_delta_body)
    d_tm, d_cm, d_ex = sc_delta(tm_AW, cm_AW, ex_AW, adapt.reshape(A))

    vrow = lambda: (0, 0)
    v, c = pl.pallas_call(
        _k2_body,
        in_specs=[
            pl.BlockSpec((D, D), vrow),
            pl.BlockSpec((D, D), vrow),
            pl.BlockSpec((E, D, D), lambda: (0, 0, 0)),
            pl.BlockSpec((D, D), vrow),
            pl.BlockSpec((D, D), vrow),
            pl.BlockSpec((D, D), vrow),
            pl.BlockSpec((D, D), vrow),
            pl.BlockSpec((E, D, D), lambda: (0, 0, 0)),
            pl.BlockSpec((E, D, D), lambda: (0, 0, 0)),
            pl.BlockSpec((E, D), vrow),
            pl.BlockSpec((1, E), vrow),
            pl.BlockSpec((1, D), vrow),
            pl.BlockSpec((1, D), vrow),
            pl.BlockSpec((D, D), vrow),
            pl.BlockSpec((1, D), vrow),
            pl.BlockSpec((1, D), vrow),
            pl.BlockSpec((1, 1), vrow),
            pl.BlockSpec((1, D), vrow),
            pl.BlockSpec((1, 1), vrow),
        ],
        out_specs=[
            pl.BlockSpec((2, D), vrow),
            pl.BlockSpec((1, 2), vrow),
        ],
        out_shape=[
            jax.ShapeDtypeStruct((2, D), jnp.float32),
            jax.ShapeDtypeStruct((1, 2), jnp.float32),
        ],
    )(d_tm.reshape(D, D), d_cm.reshape(D, D), d_ex.reshape(E, D, D),
      tm_W, tm_Ab.reshape(D, D), cm_W, cm_Ab.reshape(D, D),
      ex_W, ex_Ab.reshape(E, D, D), ex_b, gate,
      tm_b.reshape(1, D), cm_b.reshape(1, D), W_out, b_out.reshape(1, D),
      W_mem, b_mem.reshape(1, 1), W_time, b_time.reshape(1, 1))

    out = pl.pallas_call(
        _k3_body,
        grid=(N // XBLK,),
        in_specs=[
            pl.BlockSpec((XBLK, D), lambda i: (i, 0)),
            pl.BlockSpec((2, D), lambda i: (0, 0)),
            pl.BlockSpec((1, 2), lambda i: (0, 0)),
        ],
        out_specs=pl.BlockSpec((XBLK, 2), lambda i: (i, 0)),
        out_shape=jax.ShapeDtypeStruct((N, 2), jnp.float32),
    )(xf, v, c)

    mem_pred = out[:, 0].reshape(B, S)
    time_pred = out[:, 1].reshape(B, S)
    return (mem_pred, time_pred)


# speedup vs baseline: 1.1618x; 1.1618x over previous
import functools

import jax
import jax.numpy as jnp
from jax import lax
from jax.experimental import pallas as pl
from jax.experimental.pallas import tpu as pltpu
from jax.experimental.pallas import tpu_sc as plsc

B, S, D, A, E = 4, 2048, 512, 64, 4
N = B * S

XBLK = 1024
CH = 256
NW = 32
L = 16


def _k1_body(x_ref, wf_ref, bf_ref, wg_ref, bg_ref, adapt_ref, gate_ref,
             acc_ref):
    i = pl.program_id(0)

    @pl.when(i == 0)
    def _():
        acc_ref[...] = jnp.zeros_like(acc_ref)

    acc_ref[...] += jnp.sum(x_ref[...], axis=0, keepdims=True)

    @pl.when(i == pl.num_programs(0) - 1)
    def _():
        mean = acc_ref[...] * (1.0 / N)
        adapt = lax.dot_general(
            mean, wf_ref[...], (((1,), (1,)), ((), ())),
            preferred_element_type=jnp.float32,
            precision=lax.Precision.HIGHEST) + bf_ref[...]
        logits = lax.dot_general(
            adapt, wg_ref[...], (((1,), (1,)), ((), ())),
            preferred_element_type=jnp.float32,
            precision=lax.Precision.HIGHEST) + bg_ref[...]
        m = jnp.max(logits, axis=-1, keepdims=True)
        eg = jnp.exp(logits - m)
        gate_ref[...] = eg / jnp.sum(eg, axis=-1, keepdims=True)
        adapt_ref[...] = adapt


def _perm(v, idx):
    return lax.gather(
        v, idx[:, None],
        lax.GatherDimensionNumbers(offset_dims=(), collapsed_slice_dims=(0,),
                                   start_index_map=(0,)),
        (1,), mode=lax.GatherScatterMode.PROMISE_IN_BOUNDS)


def _sc_delta_body(tm_ref, cm_ref, ex_ref, adapt_ref,
                   dtm_ref, dcm_ref, dex_ref,
                   buf0, buf1, obuf, adapt_v, sem0, sem1):
    wid = lax.axis_index("s") * 2 + lax.axis_index("c")

    pltpu.sync_copy(adapt_ref, adapt_v)
    a_vs = [adapt_v[pl.ds(16 * i, L)] for i in range(4)]
    iota = lax.iota(jnp.int32, L)
    bfly = [lax.bitwise_xor(iota, k) for k in (8, 4, 2, 1)]
    masks = [iota == rr for rr in range(L)]
    zero = jnp.zeros((L,), jnp.float32)

    def segment(slicer, out_ref, out_base, rows_per_tile):
        base = wid * rows_per_tile
        nch = rows_per_tile // CH

        def compute(b, r0):
            def group_body(g, _):
                off = g * L
                out = zero
                for rr in range(L):
                    row = off + rr
                    s = (b[row, pl.ds(0, L)] * a_vs[0]
                         + b[row, pl.ds(16, L)] * a_vs[1]
                         + b[row, pl.ds(32, L)] * a_vs[2]
                         + b[row, pl.ds(48, L)] * a_vs[3])
                    for bidx in bfly:
                        s = s + _perm(s, bidx)
                    out = jnp.where(masks[rr], s, out)
                obuf[pl.ds(g * L, L)] = out
                return 0

            lax.fori_loop(0, CH // L, group_body, 0)
            pltpu.sync_copy(obuf, out_ref.at[pl.ds(out_base + r0, CH)])

        pltpu.make_async_copy(slicer(base), buf0, sem0).start()

        def pair_body(j, _):
            c0 = base + (2 * j) * CH
            c1 = c0 + CH
            pltpu.make_async_copy(slicer(c1), buf1, sem1).start()
            pltpu.make_async_copy(slicer(c0), buf0, sem0).wait()
            compute(buf0, c0)

            @pl.when(j < nch // 2 - 1)
            def _():
                pltpu.make_async_copy(slicer(c0 + 2 * CH), buf0, sem0).start()

            pltpu.make_async_copy(slicer(c1), buf1, sem1).wait()
            compute(buf1, c1)
            return 0

        lax.fori_loop(0, nch // 2, pair_body, 0)

    segment(lambda r0: tm_ref.at[pl.ds(r0, CH)], dtm_ref, 0, (D * D) // NW)
    segment(lambda r0: cm_ref.at[pl.ds(r0, CH)], dcm_ref, 0, (D * D) // NW)
    for e in range(E):
        segment(lambda r0, e=e: ex_ref.at[e, pl.ds(r0, CH)],
                dex_ref, e * D * D, (D * D) // NW)


def _k2_body(dtm_ref, dcm_ref, dex_ref,
             tmW_ref, tmAb_ref, cmW_ref, cmAb_ref,
             exW_ref, exAb_ref, exb_ref, gate_ref,
             tmb_ref, cmb_ref, wout_ref, bout_ref,
             wmem_ref, bmem_ref, wtime_ref, btime_ref,
             v_ref, c_ref):
    tmw = tmW_ref[...] + tmAb_ref[...] + dtm_ref[...]
    cmw = cmW_ref[...] + cmAb_ref[...] + dcm_ref[...]
    ew = exW_ref[...] + exAb_ref[...] + dex_ref[...]
    acc = gate_ref[0, 0] * ew[0]
    for e in range(1, E):
        acc += gate_ref[0, e] * ew[e]
    gate = gate_ref[...]
    bcomb = lax.dot_general(
        gate, exb_ref[...], (((1,), (0,)), ((), ())),
        preferred_element_type=jnp.float32,
        precision=lax.Precision.HIGHEST)

    def chain(u0, c0):
        u1 = jnp.dot(u0, wout_ref[...],
                     preferred_element_type=jnp.float32,
                     precision=lax.Precision.HIGHEST)
        c1 = c0 + jnp.sum(u0 * bout_ref[...], keepdims=True)[:, :1]
        u2 = jnp.dot(u1, acc,
                     preferred_element_type=jnp.float32,
                     precision=lax.Precision.HIGHEST)
        c2 = c1 + jnp.sum(u1 * bcomb, keepdims=True)[:, :1]
        u3 = jnp.dot(u2, cmw,
                     preferred_element_type=jnp.float32,
                     precision=lax.Precision.HIGHEST)
        c3 = c2 + jnp.sum(u2 * cmb_ref[...], keepdims=True)[:, :1]
        u4 = jnp.dot(u3, tmw,
                     preferred_element_type=jnp.float32,
                     precision=lax.Precision.HIGHEST)
        c4 = c3 + jnp.sum(u3 * tmb_ref[...], keepdims=True)[:, :1]
        return u4, c4

    vm, cm = chain(wmem_ref[...], bmem_ref[...])
    vt, ct = chain(wtime_ref[...], btime_ref[...])
    v_ref[0:1, :] = vm
    v_ref[1:2, :] = vt
    c_ref[:, 0:1] = cm
    c_ref[:, 1:2] = ct


def _k3_body(x_ref, v_ref, c_ref, out_ref):
    out_ref[...] = lax.dot_general(
        x_ref[...], v_ref[...], (((1,), (1,)), ((), ())),
        preferred_element_type=jnp.float32,
        precision=lax.Precision.HIGHEST) + c_ref[...]


@jax.jit
def kernel(x, W_feat, b_feat, tm_W, tm_b, tm_AW, tm_Ab, cm_W, cm_b, cm_AW,
           cm_Ab, ex_W, ex_b, ex_AW, ex_Ab, W_gate, b_gate, W_out, b_out,
           W_mem, b_mem, W_time, b_time):
    xf = x.reshape(N, D)

    adapt, gate = pl.pallas_call(
        _k1_body,
        grid=(N // XBLK,),
        in_specs=[
            pl.BlockSpec((XBLK, D), lambda i: (i, 0)),
            pl.BlockSpec((A, D), lambda i: (0, 0)),
            pl.BlockSpec((1, A), lambda i: (0, 0)),
            pl.BlockSpec((E, A), lambda i: (0, 0)),
            pl.BlockSpec((1, E), lambda i: (0, 0)),
        ],
        out_specs=[
            pl.BlockSpec((1, A), lambda i: (0, 0)),
            pl.BlockSpec((1, E), lambda i: (0, 0)),
        ],
        out_shape=[
            jax.ShapeDtypeStruct((1, A), jnp.float32),
            jax.ShapeDtypeStruct((1, E), jnp.float32),
        ],
        scratch_shapes=[pltpu.VMEM((1, D), jnp.float32)],
    )(xf, W_feat, b_feat.reshape(1, A), W_gate, b_gate.reshape(1, E))

    mesh = plsc.VectorSubcoreMesh(core_axis_name="c", subcore_axis_name="s")
    sc_delta = functools.partial(
        pl.kernel,
        mesh=mesh,
        out_type=[
            jax.ShapeDtypeStruct((D * D,), jnp.float32),
            jax.ShapeDtypeStruct((D * D,), jnp.float32),
            jax.ShapeDtypeStruct((E * D * D,), jnp.float32),
        ],
        scratch_types=[
            pltpu.VMEM((CH, A), jnp.float32),
            pltpu.VMEM((CH, A), jnp.float32),
            pltpu.VMEM((CH,), jnp.float32),
            pltpu.VMEM((A,), jnp.float32),
            pltpu.SemaphoreType.DMA,
            pltpu.SemaphoreType.DMA,
        ],
    )(_sc_delta_body)
    d_tm, d_cm, d_ex = sc_delta(tm_AW, cm_AW, ex_AW, adapt.reshape(A))

    vrow = lambda: (0, 0)
    v, c = pl.pallas_call(
        _k2_body,
        in_specs=[
            pl.BlockSpec((D, D), vrow),
            pl.BlockSpec((D, D), vrow),
            pl.BlockSpec((E, D, D), lambda: (0, 0, 0)),
            pl.BlockSpec((D, D), vrow),
            pl.BlockSpec((D, D), vrow),
            pl.BlockSpec((D, D), vrow),
            pl.BlockSpec((D, D), vrow),
            pl.BlockSpec((E, D, D), lambda: (0, 0, 0)),
            pl.BlockSpec((E, D, D), lambda: (0, 0, 0)),
            pl.BlockSpec((E, D), vrow),
            pl.BlockSpec((1, E), vrow),
            pl.BlockSpec((1, D), vrow),
            pl.BlockSpec((1, D), vrow),
            pl.BlockSpec((D, D), vrow),
            pl.BlockSpec((1, D), vrow),
            pl.BlockSpec((1, D), vrow),
            pl.BlockSpec((1, 1), vrow),
            pl.BlockSpec((1, D), vrow),
            pl.BlockSpec((1, 1), vrow),
        ],
        out_specs=[
            pl.BlockSpec((2, D), vrow),
            pl.BlockSpec((1, 2), vrow),
        ],
        out_shape=[
            jax.ShapeDtypeStruct((2, D), jnp.float32),
            jax.ShapeDtypeStruct((1, 2), jnp.float32),
        ],
    )(d_tm.reshape(D, D), d_cm.reshape(D, D), d_ex.reshape(E, D, D),
      tm_W, tm_Ab.reshape(D, D), cm_W, cm_Ab.reshape(D, D),
      ex_W, ex_Ab.reshape(E, D, D), ex_b, gate,
      tm_b.reshape(1, D), cm_b.reshape(1, D), W_out, b_out.reshape(1, D),
      W_mem, b_mem.reshape(1, 1), W_time, b_time.reshape(1, 1))

    out = pl.pallas_call(
        _k3_body,
        grid=(N // XBLK,),
        in_specs=[
            pl.BlockSpec((XBLK, D), lambda i: (i, 0)),
            pl.BlockSpec((2, D), lambda i: (0, 0)),
            pl.BlockSpec((1, 2), lambda i: (0, 0)),
        ],
        out_specs=pl.BlockSpec((XBLK, 2), lambda i: (i, 0)),
        out_shape=jax.ShapeDtypeStruct((N, 2), jnp.float32),
    )(xf, v, c)

    mem_pred = out[:, 0].reshape(B, S)
    time_pred = out[:, 1].reshape(B, S)
    return (mem_pred, time_pred)


# speedup vs baseline: 1.5149x; 1.3038x over previous
import functools

import jax
import jax.numpy as jnp
from jax import lax
from jax.experimental import pallas as pl
from jax.experimental.pallas import tpu as pltpu
from jax.experimental.pallas import tpu_sc as plsc

B, S, D, A, E = 4, 2048, 512, 64, 4
N = B * S

XBLK = 1024
CH = 256
NW = 32
L = 16


def _k1_body(x_ref, wf_ref, bf_ref, wg_ref, bg_ref, adapt_ref, gate_ref,
             acc_ref):
    i = pl.program_id(0)

    @pl.when(i == 0)
    def _():
        acc_ref[...] = jnp.zeros_like(acc_ref)

    acc_ref[...] += jnp.sum(x_ref[...], axis=0, keepdims=True)

    @pl.when(i == pl.num_programs(0) - 1)
    def _():
        mean = acc_ref[...] * (1.0 / N)
        adapt = lax.dot_general(
            mean, wf_ref[...], (((1,), (1,)), ((), ())),
            preferred_element_type=jnp.float32,
            precision=lax.Precision.HIGHEST) + bf_ref[...]
        logits = lax.dot_general(
            adapt, wg_ref[...], (((1,), (1,)), ((), ())),
            preferred_element_type=jnp.float32,
            precision=lax.Precision.HIGHEST) + bg_ref[...]
        m = jnp.max(logits, axis=-1, keepdims=True)
        eg = jnp.exp(logits - m)
        gate_ref[...] = eg / jnp.sum(eg, axis=-1, keepdims=True)
        adapt_ref[...] = adapt


def _perm(v, idx):
    return lax.gather(
        v, idx[:, None],
        lax.GatherDimensionNumbers(offset_dims=(), collapsed_slice_dims=(0,),
                                   start_index_map=(0,)),
        (1,), mode=lax.GatherScatterMode.PROMISE_IN_BOUNDS)


def _sc_delta_body(tm_ref, cm_ref, ex_ref, adapt_ref,
                   dtm_ref, dcm_ref, dex_ref,
                   buf0, buf1, obuf, adapt_v, sem0, sem1):
    wid = lax.axis_index("s") * 2 + lax.axis_index("c")

    pltpu.sync_copy(adapt_ref, adapt_v)
    a_vs = [adapt_v[pl.ds(16 * i, L)] for i in range(4)]
    iota = lax.iota(jnp.int32, L)
    bfly = [lax.bitwise_xor(iota, k) for k in (8, 4, 2, 1)]
    masks = [iota == rr for rr in range(L)]
    zero = jnp.zeros((L,), jnp.float32)

    def segment(slicer, out_ref, out_base, rows_per_tile):
        base = wid * rows_per_tile
        nch = rows_per_tile // CH

        def compute(b, r0):
            def group_body(g, _):
                off = g * L
                out = zero
                for rr in range(L):
                    row = off + rr
                    s = (b[row, pl.ds(0, L)] * a_vs[0]
                         + b[row, pl.ds(16, L)] * a_vs[1]
                         + b[row, pl.ds(32, L)] * a_vs[2]
                         + b[row, pl.ds(48, L)] * a_vs[3])
                    for bidx in bfly:
                        s = s + _perm(s, bidx)
                    out = jnp.where(masks[rr], s, out)
                obuf[pl.ds(g * L, L)] = out
                return 0

            lax.fori_loop(0, CH // L, group_body, 0)
            pltpu.sync_copy(obuf, out_ref.at[pl.ds(out_base + r0, CH)])

        pltpu.make_async_copy(slicer(base), buf0, sem0).start()

        def pair_body(j, _):
            c0 = base + (2 * j) * CH
            c1 = c0 + CH
            pltpu.make_async_copy(slicer(c1), buf1, sem1).start()
            pltpu.make_async_copy(slicer(c0), buf0, sem0).wait()
            compute(buf0, c0)

            @pl.when(j < nch // 2 - 1)
            def _():
                pltpu.make_async_copy(slicer(c0 + 2 * CH), buf0, sem0).start()

            pltpu.make_async_copy(slicer(c1), buf1, sem1).wait()
            compute(buf1, c1)
            return 0

        lax.fori_loop(0, nch // 2, pair_body, 0)

    segment(lambda r0: tm_ref.at[pl.ds(r0, CH)], dtm_ref, 0, (D * D) // NW)
    segment(lambda r0: cm_ref.at[pl.ds(r0, CH)], dcm_ref, 0, (D * D) // NW)
    segment(lambda r0: ex_ref.at[pl.ds(r0, CH)], dex_ref, 0,
            (E * D * D) // NW)


def _k2_body(dtm_ref, dcm_ref, dex_ref,
             tmW_ref, tmAb_ref, cmW_ref, cmAb_ref,
             exW_ref, exAb_ref, exb_ref, gate_ref,
             tmb_ref, cmb_ref, wout_ref, bout_ref,
             wmem_ref, bmem_ref, wtime_ref, btime_ref,
             v_ref, c_ref):
    tmw = tmW_ref[...] + tmAb_ref[...] + dtm_ref[...]
    cmw = cmW_ref[...] + cmAb_ref[...] + dcm_ref[...]
    ew = exW_ref[...] + exAb_ref[...] + dex_ref[...]
    acc = gate_ref[0, 0] * ew[0]
    for e in range(1, E):
        acc += gate_ref[0, e] * ew[e]
    gate = gate_ref[...]
    bcomb = lax.dot_general(
        gate, exb_ref[...], (((1,), (0,)), ((), ())),
        preferred_element_type=jnp.float32,
        precision=lax.Precision.HIGHEST)

    def chain(u0, c0):
        u1 = jnp.dot(u0, wout_ref[...],
                     preferred_element_type=jnp.float32,
                     precision=lax.Precision.HIGHEST)
        c1 = c0 + jnp.sum(u0 * bout_ref[...], keepdims=True)[:, :1]
        u2 = jnp.dot(u1, acc,
                     preferred_element_type=jnp.float32,
                     precision=lax.Precision.HIGHEST)
        c2 = c1 + jnp.sum(u1 * bcomb, keepdims=True)[:, :1]
        u3 = jnp.dot(u2, cmw,
                     preferred_element_type=jnp.float32,
                     precision=lax.Precision.HIGHEST)
        c3 = c2 + jnp.sum(u2 * cmb_ref[...], keepdims=True)[:, :1]
        u4 = jnp.dot(u3, tmw,
                     preferred_element_type=jnp.float32,
                     precision=lax.Precision.HIGHEST)
        c4 = c3 + jnp.sum(u3 * tmb_ref[...], keepdims=True)[:, :1]
        return u4, c4

    vm, cm = chain(wmem_ref[...], bmem_ref[...])
    vt, ct = chain(wtime_ref[...], btime_ref[...])
    v_ref[0:1, :] = vm
    v_ref[1:2, :] = vt
    c_ref[:, 0:1] = cm
    c_ref[:, 1:2] = ct


def _k3_body(x_ref, v_ref, c_ref, out_ref):
    out_ref[...] = lax.dot_general(
        x_ref[...], v_ref[...], (((1,), (1,)), ((), ())),
        preferred_element_type=jnp.float32,
        precision=lax.Precision.HIGHEST) + c_ref[...]


@jax.jit
def kernel(x, W_feat, b_feat, tm_W, tm_b, tm_AW, tm_Ab, cm_W, cm_b, cm_AW,
           cm_Ab, ex_W, ex_b, ex_AW, ex_Ab, W_gate, b_gate, W_out, b_out,
           W_mem, b_mem, W_time, b_time):
    xf = x.reshape(N, D)

    adapt, gate = pl.pallas_call(
        _k1_body,
        grid=(N // XBLK,),
        in_specs=[
            pl.BlockSpec((XBLK, D), lambda i: (i, 0)),
            pl.BlockSpec((A, D), lambda i: (0, 0)),
            pl.BlockSpec((1, A), lambda i: (0, 0)),
            pl.BlockSpec((E, A), lambda i: (0, 0)),
            pl.BlockSpec((1, E), lambda i: (0, 0)),
        ],
        out_specs=[
            pl.BlockSpec((1, A), lambda i: (0, 0)),
            pl.BlockSpec((1, E), lambda i: (0, 0)),
        ],
        out_shape=[
            jax.ShapeDtypeStruct((1, A), jnp.float32),
            jax.ShapeDtypeStruct((1, E), jnp.float32),
        ],
        scratch_shapes=[pltpu.VMEM((1, D), jnp.float32)],
    )(xf, W_feat, b_feat.reshape(1, A), W_gate, b_gate.reshape(1, E))

    mesh = plsc.VectorSubcoreMesh(core_axis_name="c", subcore_axis_name="s")
    sc_delta = functools.partial(
        pl.kernel,
        mesh=mesh,
        out_type=[
            jax.ShapeDtypeStruct((D * D,), jnp.float32),
            jax.ShapeDtypeStruct((D * D,), jnp.float32),
            jax.ShapeDtypeStruct((E * D * D,), jnp.float32),
        ],
        scratch_types=[
            pltpu.VMEM((CH, A), jnp.float32),
            pltpu.VMEM((CH, A), jnp.float32),
            pltpu.VMEM((CH,), jnp.float32),
            pltpu.VMEM((A,), jnp.float32),
            pltpu.SemaphoreType.DMA,
            pltpu.SemaphoreType.DMA,
        ],
    )(_sc_delta_body)
    d_tm, d_cm, d_ex = sc_delta(
        tm_AW, cm_AW, ex_AW.reshape(E * D * D, A), adapt.reshape(A))

    vrow = lambda: (0, 0)
    v, c = pl.pallas_call(
        _k2_body,
        in_specs=[
            pl.BlockSpec((D, D), vrow),
            pl.BlockSpec((D, D), vrow),
            pl.BlockSpec((E, D, D), lambda: (0, 0, 0)),
            pl.BlockSpec((D, D), vrow),
            pl.BlockSpec((D, D), vrow),
            pl.BlockSpec((D, D), vrow),
            pl.BlockSpec((D, D), vrow),
            pl.BlockSpec((E, D, D), lambda: (0, 0, 0)),
            pl.BlockSpec((E, D, D), lambda: (0, 0, 0)),
            pl.BlockSpec((E, D), vrow),
            pl.BlockSpec((1, E), vrow),
            pl.BlockSpec((1, D), vrow),
            pl.BlockSpec((1, D), vrow),
            pl.BlockSpec((D, D), vrow),
            pl.BlockSpec((1, D), vrow),
            pl.BlockSpec((1, D), vrow),
            pl.BlockSpec((1, 1), vrow),
            pl.BlockSpec((1, D), vrow),
            pl.BlockSpec((1, 1), vrow),
        ],
        out_specs=[
            pl.BlockSpec((2, D), vrow),
            pl.BlockSpec((1, 2), vrow),
        ],
        out_shape=[
            jax.ShapeDtypeStruct((2, D), jnp.float32),
            jax.ShapeDtypeStruct((1, 2), jnp.float32),
        ],
    )(d_tm.reshape(D, D), d_cm.reshape(D, D), d_ex.reshape(E, D, D),
      tm_W, tm_Ab.reshape(D, D), cm_W, cm_Ab.reshape(D, D),
      ex_W, ex_Ab.reshape(E, D, D), ex_b, gate,
      tm_b.reshape(1, D), cm_b.reshape(1, D), W_out, b_out.reshape(1, D),
      W_mem, b_mem.reshape(1, 1), W_time, b_time.reshape(1, 1))

    out = pl.pallas_call(
        _k3_body,
        grid=(N // XBLK,),
        in_specs=[
            pl.BlockSpec((XBLK, D), lambda i: (i, 0)),
            pl.BlockSpec((2, D), lambda i: (0, 0)),
            pl.BlockSpec((1, 2), lambda i: (0, 0)),
        ],
        out_specs=pl.BlockSpec((XBLK, 2), lambda i: (i, 0)),
        out_shape=jax.ShapeDtypeStruct((N, 2), jnp.float32),
    )(xf, v, c)

    mem_pred = out[:, 0].reshape(B, S)
    time_pred = out[:, 1].reshape(B, S)
    return (mem_pred, time_pred)
